# baseline (speedup 1.0000x reference)
import functools

import jax
import jax.numpy as jnp
from jax import lax
from jax.experimental import pallas as pl
from jax.experimental.pallas import tpu as pltpu
from jax.experimental.pallas import tpu_sc as plsc

BATCH = 1024
SEQ_LEN = 200
EMB_DIM = 32
OUT_ROWS = SEQ_LEN + 1
HALF = SEQ_LEN // 2
GROUP = 8

_info = plsc.get_sparse_core_info()
_NC, _NS = _info.num_cores, _info.num_subcores
NW = _NC * _NS
BPW = BATCH // NW
NGROUP = BPW // GROUP
BLOCK_ROWS = GROUP * OUT_ROWS


@functools.partial(
    pl.kernel,
    mesh=plsc.VectorSubcoreMesh(core_axis_name="c", subcore_axis_name="s"),
    out_type=jax.ShapeDtypeStruct((BATCH * OUT_ROWS, EMB_DIM), jnp.float32),
    scratch_types=[
        pltpu.VMEM((2 * BPW, HALF), jnp.int32),
        pltpu.VMEM((2 * BLOCK_ROWS, EMB_DIM), jnp.float32),
        pltpu.SemaphoreType.DMA,
        pltpu.SemaphoreType.DMA,
    ],
    compiler_params=pltpu.CompilerParams(use_tc_tiling_on_sc=False),
)
def _emb_kernel(idx_hbm, table_hbm, empty_hbm, out_hbm,
                idx_v, rows_v, sem_g, sem_s):
    wid = lax.axis_index("s") * _NC + lax.axis_index("c")
    base = wid * BPW
    pltpu.sync_copy(idx_hbm.at[pl.ds(2 * base, 2 * BPW)], idx_v)
    for d in range(2):
        for j in range(GROUP):
            pltpu.sync_copy(
                empty_hbm,
                rows_v.at[pl.ds(d * BLOCK_ROWS + j * OUT_ROWS, 1)])

    def gather_group(g, d):
        for j in range(GROUP):
            for h in range(2):
                pltpu.async_copy(
                    table_hbm.at[idx_v.at[2 * (g * GROUP + j) + h]],
                    rows_v.at[pl.ds(d * BLOCK_ROWS + j * OUT_ROWS + 1 + h * HALF,
                                    HALF)],
                    sem_g)

    def drain_gathers(g, d):
        for j in range(GROUP):
            for h in range(2):
                pltpu.make_async_copy(
                    table_hbm.at[idx_v.at[2 * (g * GROUP + j) + h]],
                    rows_v.at[pl.ds(d * BLOCK_ROWS + j * OUT_ROWS + 1 + h * HALF,
                                    HALF)],
                    sem_g).wait()

    def store_desc(g, d):
        return pltpu.make_async_copy(
            rows_v.at[pl.ds(d * BLOCK_ROWS, BLOCK_ROWS)],
            out_hbm.at[pl.ds((base + g * GROUP) * OUT_ROWS, BLOCK_ROWS)],
            sem_s)

    gather_group(0, 0)

    def body(g, _):
        d = g % 2
        drain_gathers(g, d)

        @pl.when(g >= 1)
        def _():
            store_desc(g - 1, 1 - d).wait()

        store_desc(g, d).start()

        @pl.when(g + 1 < NGROUP)
        def _():
            gather_group(g + 1, 1 - d)

        return 0

    lax.fori_loop(0, NGROUP, body, 0)
    store_desc(NGROUP - 1, (NGROUP - 1) % 2).wait()


def kernel(sentence, table, empty_emb):
    idx = sentence[:, ::-1].astype(jnp.int32).reshape(2 * BATCH, HALF)
    flat = _emb_kernel(idx, table, empty_emb)
    return flat.reshape(BATCH, OUT_ROWS, EMB_DIM)


# speedup vs baseline: 1.0029x; 1.0029x over previous
import functools

import jax
import jax.numpy as jnp
from jax import lax
from jax.experimental import pallas as pl
from jax.experimental.pallas import tpu as pltpu
from jax.experimental.pallas import tpu_sc as plsc

BATCH = 1024
SEQ_LEN = 200
EMB_DIM = 32
OUT_ROWS = SEQ_LEN + 1
CHUNKS = (104, 96)
GROUP = 8

_info = plsc.get_sparse_core_info()
_NC, _NS = _info.num_cores, _info.num_subcores
NW = _NC * _NS
BPW = BATCH // NW
NGROUP = BPW // GROUP
BLOCK_ROWS = GROUP * OUT_ROWS


@functools.partial(
    pl.kernel,
    mesh=plsc.VectorSubcoreMesh(core_axis_name="c", subcore_axis_name="s"),
    out_type=jax.ShapeDtypeStruct((BATCH * OUT_ROWS, EMB_DIM), jnp.float32),
    scratch_types=[
        pltpu.VMEM((BPW * SEQ_LEN,), jnp.int32),
        pltpu.VMEM((2 * BLOCK_ROWS, EMB_DIM), jnp.float32),
        pltpu.SemaphoreType.DMA,
        pltpu.SemaphoreType.DMA,
    ],
    compiler_params=pltpu.CompilerParams(use_tc_tiling_on_sc=False),
)
def _emb_kernel(idx_hbm, table_hbm, empty_hbm, out_hbm,
                idx_v, rows_v, sem_g, sem_s):
    wid = lax.axis_index("s") * _NC + lax.axis_index("c")
    base = wid * BPW
    pltpu.sync_copy(idx_hbm.at[pl.ds(base * SEQ_LEN, BPW * SEQ_LEN)], idx_v)
    for d in range(2):
        for j in range(GROUP):
            pltpu.sync_copy(
                empty_hbm,
                rows_v.at[pl.ds(d * BLOCK_ROWS + j * OUT_ROWS, 1)])

    def gather_copies(g, d, make_only):
        for j in range(GROUP):
            off = 0
            for n in CHUNKS:
                desc = pltpu.make_async_copy(
                    table_hbm.at[idx_v.at[pl.ds((g * GROUP + j) * SEQ_LEN
                                                + off, n)]],
                    rows_v.at[pl.ds(d * BLOCK_ROWS + j * OUT_ROWS + 1 + off, n)],
                    sem_g)
                if make_only:
                    desc.wait()
                else:
                    desc.start()
                off += n

    def gather_group(g, d):
        gather_copies(g, d, make_only=False)

    def drain_gathers(g, d):
        gather_copies(g, d, make_only=True)

    def store_desc(g, d):
        return pltpu.make_async_copy(
            rows_v.at[pl.ds(d * BLOCK_ROWS, BLOCK_ROWS)],
            out_hbm.at[pl.ds((base + g * GROUP) * OUT_ROWS, BLOCK_ROWS)],
            sem_s)

    gather_group(0, 0)

    def body(g, _):
        d = g % 2
        drain_gathers(g, d)

        @pl.when(g >= 1)
        def _():
            store_desc(g - 1, 1 - d).wait()

        store_desc(g, d).start()

        @pl.when(g + 1 < NGROUP)
        def _():
            gather_group(g + 1, 1 - d)

        return 0

    lax.fori_loop(0, NGROUP, body, 0)
    store_desc(NGROUP - 1, (NGROUP - 1) % 2).wait()


def kernel(sentence, table, empty_emb):
    idx = sentence[:, ::-1].astype(jnp.int32).reshape(-1)
    flat = _emb_kernel(idx, table, empty_emb)
    return flat.reshape(BATCH, OUT_ROWS, EMB_DIM)


# speedup vs baseline: 1.1701x; 1.1667x over previous
import functools

import jax
import jax.numpy as jnp
from jax import lax
from jax.experimental import pallas as pl
from jax.experimental.pallas import tpu as pltpu
from jax.experimental.pallas import tpu_sc as plsc

BATCH = 1024
SEQ_LEN = 200
EMB_DIM = 32
OUT_ROWS = SEQ_LEN + 1
CHUNKS = (104, 96)
GROUP = 8

_info = plsc.get_sparse_core_info()
_NC, _NS = _info.num_cores, _info.num_subcores
NW = _NC * _NS
BPW = BATCH // NW
NGROUP = BPW // GROUP
BLOCK_ROWS = GROUP * OUT_ROWS


@functools.partial(
    pl.kernel,
    mesh=plsc.VectorSubcoreMesh(core_axis_name="c", subcore_axis_name="s"),
    out_type=jax.ShapeDtypeStruct((BATCH, OUT_ROWS, EMB_DIM), jnp.float32),
    scratch_types=[
        pltpu.VMEM((BPW * SEQ_LEN,), jnp.int32),
        pltpu.VMEM((2 * GROUP, OUT_ROWS, EMB_DIM), jnp.float32),
        pltpu.SemaphoreType.DMA,
        pltpu.SemaphoreType.DMA,
    ],
    compiler_params=pltpu.CompilerParams(use_tc_tiling_on_sc=False),
)
def _emb_kernel(idx_hbm, table_hbm, empty_hbm, out_hbm,
                idx_v, rows_v, sem_g, sem_s):
    wid = lax.axis_index("s") * _NC + lax.axis_index("c")
    base = wid * BPW
    pltpu.sync_copy(idx_hbm.at[pl.ds(base * SEQ_LEN, BPW * SEQ_LEN)], idx_v)
    for d in range(2):
        for j in range(GROUP):
            pltpu.sync_copy(
                empty_hbm, rows_v.at[d * GROUP + j, pl.ds(0, 1)])

    def gather_copies(g, d, make_only):
        for j in range(GROUP):
            off = 0
            for n in CHUNKS:
                desc = pltpu.make_async_copy(
                    table_hbm.at[idx_v.at[pl.ds((g * GROUP + j) * SEQ_LEN
                                                + off, n)]],
                    rows_v.at[d * GROUP + j, pl.ds(1 + off, n)],
                    sem_g)
                if make_only:
                    desc.wait()
                else:
                    desc.start()
                off += n

    def gather_group(g, d):
        gather_copies(g, d, make_only=False)

    def drain_gathers(g, d):
        gather_copies(g, d, make_only=True)

    def store_desc(g, d):
        return pltpu.make_async_copy(
            rows_v.at[pl.ds(d * GROUP, GROUP)],
            out_hbm.at[pl.ds(base + g * GROUP, GROUP)],
            sem_s)

    gather_group(0, 0)

    def body(g, _):
        d = g % 2
        drain_gathers(g, d)

        @pl.when(g >= 1)
        def _():
            store_desc(g - 1, 1 - d).wait()

        store_desc(g, d).start()

        @pl.when(g + 1 < NGROUP)
        def _():
            gather_group(g + 1, 1 - d)

        return 0

    lax.fori_loop(0, NGROUP, body, 0)
    store_desc(NGROUP - 1, (NGROUP - 1) % 2).wait()


def kernel(sentence, table, empty_emb):
    idx = sentence[:, ::-1].astype(jnp.int32).reshape(-1)
    return _emb_kernel(idx, table, empty_emb)
